# baseline (speedup 1.0000x reference)
import jax
import jax.numpy as jnp
from jax.experimental import pallas as pl

NUM_CLASSES = 80
K = 300


def _pre_kernel(logits_ref, boxes_ref, sizes_ref, scores_ref, bbox_ref):
    scores_ref[...] = jax.nn.sigmoid(logits_ref[...])
    b = boxes_ref[...]
    lo = b[..., :2] - 0.5 * b[..., 2:]
    hi = b[..., :2] + 0.5 * b[..., 2:]
    xyxy = jnp.concatenate([lo, hi], axis=-1)
    s = sizes_ref[...]
    bbox_ref[...] = xyxy * s[:, None, :]


def kernel(pred_logits, pred_boxes, orig_target_sizes):
    B, Q, C = pred_logits.shape
    sizes4 = jnp.tile(orig_target_sizes.astype(jnp.float32), (1, 2))
    bb = 8
    scores, bbox = pl.pallas_call(
        _pre_kernel,
        grid=(B // bb,),
        in_specs=[
            pl.BlockSpec((bb, Q, C), lambda i: (i, 0, 0)),
            pl.BlockSpec((bb, Q, 4), lambda i: (i, 0, 0)),
            pl.BlockSpec((bb, 4), lambda i: (i, 0)),
        ],
        out_specs=[
            pl.BlockSpec((bb, Q, C), lambda i: (i, 0, 0)),
            pl.BlockSpec((bb, Q, 4), lambda i: (i, 0, 0)),
        ],
        out_shape=[
            jax.ShapeDtypeStruct((B, Q, C), jnp.float32),
            jax.ShapeDtypeStruct((B, Q, 4), jnp.float32),
        ],
    )(pred_logits, pred_boxes, sizes4)
    flat = scores.reshape(B, Q * C)
    top_scores, index = jax.lax.top_k(flat, K)
    labels = index % C
    box_index = index // C
    boxes = jnp.take_along_axis(bbox, box_index[..., None], axis=1)
    return (labels, boxes, top_scores)


# speedup vs baseline: 7.8728x; 7.8728x over previous
import functools

import jax
import jax.numpy as jnp
from jax import lax
from jax.experimental import pallas as pl
from jax.experimental.pallas import tpu as pltpu
from jax.experimental.pallas import tpu_sc as plsc

NUM_CLASSES = 80
K = 300
B = 64
NQ = 900
N = NQ * NUM_CLASSES
NV = N // 16
KPAD = 304
CAP = 4096

_i32 = jnp.int32
_f32 = jnp.float32
_SIGN = -2147483648


def _monotonic_key(bits):
    m = lax.shift_right_arithmetic(bits, jnp.full(bits.shape, 31, _i32))
    return bits ^ (m | jnp.full(bits.shape, _SIGN, _i32))


def _shr_l(x, amt):
    return lax.shift_right_logical(x, jnp.full(x.shape, amt, _i32))


def _build_sc_call():
    nc, ns = 2, 16
    nw = nc * ns
    rows_per_w = B // nw
    mesh = plsc.VectorSubcoreMesh(core_axis_name="c", subcore_axis_name="s",
                                  num_cores=nc, num_subcores=ns)

    @functools.partial(
        pl.kernel,
        mesh=mesh,
        compiler_params=pltpu.CompilerParams(needs_layout_passes=False),
        out_type=[
            jax.ShapeDtypeStruct((B, KPAD), _i32),
            jax.ShapeDtypeStruct((B, KPAD), _f32),
            jax.ShapeDtypeStruct((B, KPAD * 4), _f32),
        ],
        scratch_types=[
            pltpu.VMEM((N,), _f32),
            pltpu.VMEM((NQ * 4,), _f32),
            pltpu.VMEM((16,), _f32),
            pltpu.VMEM((4096,), _i32),
            pltpu.VMEM((CAP,), _f32),
            pltpu.VMEM((CAP,), _i32),
            pltpu.VMEM((CAP,), _i32),
            pltpu.VMEM((CAP,), _i32),
            pltpu.VMEM((CAP,), _i32),
            pltpu.VMEM((512,), _i32),
            pltpu.VMEM((KPAD,), _i32),
            pltpu.VMEM((KPAD,), _f32),
            pltpu.VMEM((KPAD * 4,), _f32),
        ],
    )
    def sc_call(logits_hbm, boxes_hbm, scale_hbm, lab_hbm, sc_hbm, box_hbm,
                data_v, boxr_v, scale_v, hist_v, cand_v,
                skey_a, skey_b, sidx_a, sidx_b, cnt_v,
                labo_v, sco_v, boxo_v):
        wid = lax.axis_index("s") * nc + lax.axis_index("c")
        lanes = lax.iota(_i32, 16)
        ones = jnp.ones((16,), _i32)
        zeros16 = jnp.zeros((16,), _i32)

        for rr in range(rows_per_w):
            r = wid * rows_per_w + rr
            pltpu.sync_copy(logits_hbm.at[r], data_v)
            pltpu.sync_copy(boxes_hbm.at[r], boxr_v)
            pltpu.sync_copy(scale_hbm.at[r], scale_v)

            def clr_hist(i, _):
                hist_v[pl.ds(i * 16, 16)] = zeros16
                return 0
            lax.fori_loop(0, 256, clr_hist, 0)

            def hist_body(i, _):
                x = data_v[pl.ds(i * 16, 16)]
                key = _monotonic_key(lax.bitcast_convert_type(x, _i32))
                bucket = _shr_l(key, 24)
                plsc.addupdate_scatter(hist_v, [bucket * 16 + lanes], ones)
                return 0
            lax.fori_loop(0, NV, hist_body, 0)

            def scan_body(i, carry):
                cum, bstar = carry
                bck = 255 - i
                tot = jnp.sum(hist_v[pl.ds(bck * 16, 16)])
                newcum = cum + tot
                found = jnp.logical_and(cum < K, newcum >= K)
                bstar = jnp.where(found, bck, bstar)
                return (newcum, bstar)
            _, bstar = lax.fori_loop(0, 256, scan_body, (_i32(0), _i32(0)))

            t = lax.shift_left(bstar, _i32(24))
            fbits = jnp.where(bstar >= 128, t & _i32(0x7FFFFFFF), ~t)
            thr = lax.bitcast_convert_type(lax.broadcast_in_dim(fbits, (16,), ()), _f32)

            def comp_body(i, off):
                x = data_v[pl.ds(i * 16, 16)]
                m = x >= thr
                guard = off <= CAP - 16
                m = jnp.logical_and(m, lax.broadcast_in_dim(guard, (16,), ()))
                soff = jnp.minimum(off, _i32(CAP - 16))
                plsc.store_compressed(cand_v.at[pl.ds(soff, 16)], x, mask=m)
                plsc.store_compressed(sidx_a.at[pl.ds(soff, 16)], lanes + i * 16, mask=m)
                return off + jnp.max(plsc.all_reduce_population_count(m))
            n_d = lax.fori_loop(0, NV, comp_body, _i32(0))
            n_d = jnp.minimum(n_d, _i32(CAP))
            nvd = (n_d + 15) // 16

            def conv_body(j, _):
                x = cand_v[pl.ds(j * 16, 16)]
                ki = ~_monotonic_key(lax.bitcast_convert_type(x, _i32))
                valid = (j * 16 + lanes) < n_d
                skey_a[pl.ds(j * 16, 16)] = jnp.where(valid, ki, _i32(-1))
                iv = sidx_a[pl.ds(j * 16, 16)]
                sidx_a[pl.ds(j * 16, 16)] = jnp.where(valid, iv, _i32(0))
                return 0
            lax.fori_loop(0, nvd, conv_body, 0)

            bufs = [(skey_a, sidx_a), (skey_b, sidx_b)]
            for p in range(7):
                src_k, src_i = bufs[p % 2]
                dst_k, dst_i = bufs[(p + 1) % 2]
                shift = 5 * p

                def clr_cnt(i, _):
                    cnt_v[pl.ds(i * 16, 16)] = zeros16
                    return 0
                lax.fori_loop(0, 32, clr_cnt, 0)

                def rs_hist(j, _):
                    g = lanes * nvd + j
                    kv = plsc.load_gather(src_k, [g])
                    d = _shr_l(kv, shift) & 31
                    plsc.addupdate_scatter(cnt_v, [d * 16 + lanes], ones)
                    return 0
                lax.fori_loop(0, nvd, rs_hist, 0)

                def rs_base(d, carry):
                    v = cnt_v[pl.ds(d * 16, 16)]
                    inc = plsc.cumsum(v)
                    cnt_v[pl.ds(d * 16, 16)] = inc - v + carry
                    return carry + jnp.max(inc)
                lax.fori_loop(0, 32, rs_base, _i32(0))

                def rs_perm(j, _):
                    g = lanes * nvd + j
                    kv = plsc.load_gather(src_k, [g])
                    iv = plsc.load_gather(src_i, [g])
                    d = _shr_l(kv, shift) & 31
                    ci = d * 16 + lanes
                    pos = plsc.load_gather(cnt_v, [ci])
                    plsc.store_scatter(cnt_v, [ci], pos + 1)
                    plsc.store_scatter(dst_k, [pos], kv)
                    plsc.store_scatter(dst_i, [pos], iv)
                    return 0
                lax.fori_loop(0, nvd, rs_perm, 0)

            sc_row = scale_v[pl.ds(0, 16)]
            sxs = jnp.sum(jnp.where(lanes == 0, sc_row, 0.0))
            sys_ = jnp.sum(jnp.where(lanes == 1, sc_row, 0.0))
            sx = lax.broadcast_in_dim(sxs, (16,), ())
            sy = lax.broadcast_in_dim(sys_, (16,), ())

            def out_body(j, _):
                ki = skey_b[pl.ds(j * 16, 16)]
                key = ~ki
                m = lax.shift_right_arithmetic(key, jnp.full((16,), 31, _i32))
                bits = key ^ (~m | _SIGN)
                x = lax.bitcast_convert_type(bits, _f32)
                score = 1.0 / (1.0 + jnp.exp(-x))
                idxv = sidx_b[pl.ds(j * 16, 16)]
                q = idxv // NUM_CLASSES
                label = idxv - q * NUM_CLASSES
                q = jnp.minimum(jnp.maximum(q, 0), NQ - 1)
                qq = q * 4
                cx = plsc.load_gather(boxr_v, [qq])
                cy = plsc.load_gather(boxr_v, [qq + 1])
                w = plsc.load_gather(boxr_v, [qq + 2])
                h = plsc.load_gather(boxr_v, [qq + 3])
                labo_v[pl.ds(j * 16, 16)] = label
                sco_v[pl.ds(j * 16, 16)] = score
                ob = (j * 16 + lanes) * 4
                plsc.store_scatter(boxo_v, [ob], (cx - 0.5 * w) * sx)
                plsc.store_scatter(boxo_v, [ob + 1], (cy - 0.5 * h) * sy)
                plsc.store_scatter(boxo_v, [ob + 2], (cx + 0.5 * w) * sx)
                plsc.store_scatter(boxo_v, [ob + 3], (cy + 0.5 * h) * sy)
                return 0
            lax.fori_loop(0, KPAD // 16, out_body, 0)

            pltpu.sync_copy(labo_v, lab_hbm.at[r])
            pltpu.sync_copy(sco_v, sc_hbm.at[r])
            pltpu.sync_copy(boxo_v, box_hbm.at[r])

    return sc_call


_SC_CALL = None


def kernel(pred_logits, pred_boxes, orig_target_sizes):
    global _SC_CALL
    if _SC_CALL is None:
        _SC_CALL = _build_sc_call()
    logits2 = pred_logits.reshape(B, N)
    boxes2 = pred_boxes.reshape(B, NQ * 4)
    s4 = jnp.tile(orig_target_sizes.astype(_f32), (1, 2))
    scale16 = jnp.concatenate([s4, jnp.zeros((B, 12), _f32)], axis=1)
    lab_p, sc_p, box_p = _SC_CALL(logits2, boxes2, scale16)
    labels = lab_p[:, :K]
    scores = sc_p[:, :K]
    boxes = box_p.reshape(B, KPAD, 4)[:, :K, :]
    return (labels, boxes, scores)


# speedup vs baseline: 8.4121x; 1.0685x over previous
import functools

import jax
import jax.numpy as jnp
from jax import lax
from jax.experimental import pallas as pl
from jax.experimental.pallas import tpu as pltpu
from jax.experimental.pallas import tpu_sc as plsc

NUM_CLASSES = 80
K = 300
B = 64
NQ = 900
N = NQ * NUM_CLASSES
MROW = 568
NPAD = MROW * 128
BOXM = 32
KPAD = 304
OUTM = 8
BOXOM = 16
CAP = 4096

_i32 = jnp.int32
_f32 = jnp.float32
_SIGN = -2147483648


def _monotonic_key(bits):
    m = lax.shift_right_arithmetic(bits, jnp.full(bits.shape, 31, _i32))
    return bits ^ (m | jnp.full(bits.shape, _SIGN, _i32))


def _shr_l(x, amt):
    return lax.shift_right_logical(x, jnp.full(x.shape, amt, _i32))


def _build_sc_call():
    nc, ns = 2, 16
    nw = nc * ns
    rows_per_w = B // nw
    mesh = plsc.VectorSubcoreMesh(core_axis_name="c", subcore_axis_name="s",
                                  num_cores=nc, num_subcores=ns)

    @functools.partial(
        pl.kernel,
        mesh=mesh,
        compiler_params=pltpu.CompilerParams(needs_layout_passes=False),
        out_type=[
            jax.ShapeDtypeStruct((B, OUTM, 128), _i32),
            jax.ShapeDtypeStruct((B, OUTM, 128), _f32),
            jax.ShapeDtypeStruct((B, BOXOM, 128), _f32),
        ],
        scratch_types=[
            pltpu.VMEM((MROW, 128), _f32),
            pltpu.VMEM((BOXM, 128), _f32),
            pltpu.VMEM((4 * 4096,), _i32),
            pltpu.VMEM((CAP,), _i32),
            pltpu.VMEM((CAP,), _i32),
            pltpu.VMEM((CAP,), _i32),
            pltpu.VMEM((CAP,), _i32),
            pltpu.VMEM((512,), _i32),
            pltpu.VMEM((OUTM, 128), _i32),
            pltpu.VMEM((OUTM, 128), _f32),
            pltpu.VMEM((BOXOM, 128), _f32),
        ],
    )
    def sc_call(logits_hbm, boxes_hbm, lab_hbm, sc_hbm, box_hbm,
                data_v, boxr_v, hist_v,
                skey_a, skey_b, sidx_a, sidx_b, cnt_v,
                labo_v, sco_v, boxo_v):
        wid = lax.axis_index("s") * nc + lax.axis_index("c")
        lanes = lax.iota(_i32, 16)
        ones = jnp.ones((16,), _i32)
        zeros16 = jnp.zeros((16,), _i32)

        def row_body(rr, _unused):
            r = wid * rows_per_w + rr
            pltpu.sync_copy(logits_hbm.at[r], data_v)
            pltpu.sync_copy(boxes_hbm.at[r], boxr_v)

            def clr_hist(i, _):
                for k in range(8):
                    hist_v[pl.ds(i * 128 + k * 16, 16)] = zeros16
                return 0
            lax.fori_loop(0, 128, clr_hist, 0)

            def hist_body(i, _):
                row = data_v.at[i]
                for k in range(8):
                    x = row[pl.ds(k * 16, 16)]
                    key = _monotonic_key(lax.bitcast_convert_type(x, _i32))
                    bucket = _shr_l(key, 24)
                    hidx = bucket * 16 + lanes + (k % 4) * 4096
                    plsc.addupdate_scatter(hist_v, [hidx], ones)
                return 0
            lax.fori_loop(0, MROW, hist_body, 0)

            def merge_hist(i, _):
                for k in range(4):
                    b = i * 4 + k
                    h = (hist_v[pl.ds(b * 16, 16)]
                         + hist_v[pl.ds(4096 + b * 16, 16)]
                         + hist_v[pl.ds(8192 + b * 16, 16)]
                         + hist_v[pl.ds(12288 + b * 16, 16)])
                    hist_v[pl.ds(b * 16, 16)] = h
                return 0
            lax.fori_loop(0, 64, merge_hist, 0)

            def scan_grp(i, carry):
                cum, gstar, gbase = carry
                g = 15 - i
                acc = hist_v[pl.ds(g * 256, 16)]
                for k in range(1, 16):
                    acc = acc + hist_v[pl.ds(g * 256 + k * 16, 16)]
                sg = jnp.sum(acc)
                newcum = cum + sg
                found = jnp.logical_and(cum < K, newcum >= K)
                gstar = jnp.where(found, g, gstar)
                gbase = jnp.where(found, cum, gbase)
                return (newcum, gstar, gbase)
            _, gstar, gbase = lax.fori_loop(
                0, 16, scan_grp, (_i32(0), _i32(0), _i32(0)))

            def scan_bck(i, carry):
                cum, bstar = carry
                b = gstar * 16 + 15 - i
                tot = jnp.sum(hist_v[pl.ds(b * 16, 16)])
                newcum = cum + tot
                found = jnp.logical_and(cum < K, newcum >= K)
                bstar = jnp.where(found, b, bstar)
                return (newcum, bstar)
            _, bstar = lax.fori_loop(0, 16, scan_bck, (gbase, _i32(0)))

            t = lax.shift_left(bstar, _i32(24))
            fbits = jnp.where(bstar >= 128, t & _i32(0x7FFFFFFF), ~t)
            thr = lax.bitcast_convert_type(
                lax.broadcast_in_dim(fbits, (16,), ()), _f32)
            capv = jnp.full((16,), CAP, _i32)

            def comp_body(i, off_vec):
                row = data_v.at[i]
                base = i * 128
                for k in range(8):
                    x = row[pl.ds(k * 16, 16)]
                    m = x >= thr
                    mi = jnp.where(m, ones, zeros16)
                    inc = plsc.cumsum(mi)
                    pos = off_vec + inc - mi
                    mm = jnp.logical_and(m, pos < capv)
                    plsc.store_scatter(sidx_a, [pos],
                                       base + k * 16 + lanes, mask=mm)
                    off_vec = off_vec + plsc.all_reduce_population_count(m)
                return off_vec
            off_vec = lax.fori_loop(0, MROW, comp_body,
                                    jnp.zeros((16,), _i32))
            n_d = jnp.minimum(jnp.max(off_vec), _i32(CAP))
            nvd = ((n_d + 63) // 64) * 4

            def conv_body(jo, _):
                for kk in range(4):
                    j = jo * 4 + kk
                    iv = sidx_a[pl.ds(j * 16, 16)]
                    valid = (j * 16 + lanes) < n_d
                    ivs = jnp.where(valid, iv, 0)
                    xr = _shr_l(ivs, 7)
                    xc = ivs & 127
                    x = plsc.load_gather(data_v, [xr, xc])
                    ki = ~_monotonic_key(lax.bitcast_convert_type(x, _i32))
                    skey_a[pl.ds(j * 16, 16)] = jnp.where(valid, ki, _i32(-1))
                    sidx_a[pl.ds(j * 16, 16)] = ivs
                return 0
            lax.fori_loop(0, nvd // 4, conv_body, 0)

            bufs = [(skey_a, sidx_a), (skey_b, sidx_b)]
            for p in range(7):
                src_k, src_i = bufs[p % 2]
                dst_k, dst_i = bufs[(p + 1) % 2]
                shift = 5 * p

                def clr_cnt(i, _):
                    for k in range(4):
                        cnt_v[pl.ds(i * 64 + k * 16, 16)] = zeros16
                    return 0
                lax.fori_loop(0, 8, clr_cnt, 0)

                def rs_hist(jo, _):
                    for kk in range(4):
                        j = jo * 4 + kk
                        g = lanes * nvd + j
                        kv = plsc.load_gather(src_k, [g])
                        d = _shr_l(kv, shift) & 31
                        plsc.addupdate_scatter(cnt_v, [d * 16 + lanes], ones)
                    return 0
                lax.fori_loop(0, nvd // 4, rs_hist, 0)

                acc0 = jnp.zeros((16,), _i32)
                acc1 = jnp.zeros((16,), _i32)
                for k in range(16):
                    acc0 = acc0 + plsc.load_gather(cnt_v, [lanes * 16 + k])
                    acc1 = acc1 + plsc.load_gather(cnt_v,
                                                   [(lanes + 16) * 16 + k])
                c0 = plsc.cumsum(acc0)
                base0 = c0 - acc0
                tot0 = jnp.max(c0)
                c1 = plsc.cumsum(acc1)
                base1 = c1 - acc1 + tot0

                def rs_lanepfx(do, _):
                    for kk in range(4):
                        d = do * 4 + kk
                        v = cnt_v[pl.ds(d * 16, 16)]
                        cnt_v[pl.ds(d * 16, 16)] = plsc.cumsum(v) - v
                    return 0
                lax.fori_loop(0, 8, rs_lanepfx, 0)
                for k in range(16):
                    plsc.addupdate_scatter(cnt_v, [lanes * 16 + k], base0)
                    plsc.addupdate_scatter(cnt_v, [(lanes + 16) * 16 + k],
                                           base1)

                def rs_perm(jo, _):
                    for kk in range(4):
                        j = jo * 4 + kk
                        g = lanes * nvd + j
                        kv = plsc.load_gather(src_k, [g])
                        iv = plsc.load_gather(src_i, [g])
                        d = _shr_l(kv, shift) & 31
                        ci = d * 16 + lanes
                        pos = plsc.load_gather(cnt_v, [ci])
                        plsc.store_scatter(cnt_v, [ci], pos + 1)
                        plsc.store_scatter(dst_k, [pos], kv)
                        plsc.store_scatter(dst_i, [pos], iv)
                    return 0
                lax.fori_loop(0, nvd // 4, rs_perm, 0)

            sc_row = boxr_v.at[28][pl.ds(16, 16)]
            sxs = jnp.sum(jnp.where(lanes == 0, sc_row, 0.0))
            sys_ = jnp.sum(jnp.where(lanes == 1, sc_row, 0.0))
            sx = lax.broadcast_in_dim(sxs, (16,), ())
            sy = lax.broadcast_in_dim(sys_, (16,), ())

            def out_body(j, _):
                ki = skey_b[pl.ds(j * 16, 16)]
                key = ~ki
                m = lax.shift_right_arithmetic(key, jnp.full((16,), 31, _i32))
                bits = key ^ (~m | jnp.full((16,), _SIGN, _i32))
                x = lax.bitcast_convert_type(bits, _f32)
                score = 1.0 / (1.0 + jnp.exp(-x))
                idxv = sidx_b[pl.ds(j * 16, 16)]
                q = idxv // NUM_CLASSES
                label = idxv - q * NUM_CLASSES
                q = jnp.minimum(jnp.maximum(q, 0), NQ - 1)
                qq = q * 4
                qr = _shr_l(qq, 7)
                qc = qq & 127
                cx = plsc.load_gather(boxr_v, [qr, qc])
                cy = plsc.load_gather(boxr_v, [qr, qc + 1])
                w = plsc.load_gather(boxr_v, [qr, qc + 2])
                h = plsc.load_gather(boxr_v, [qr, qc + 3])
                pv = j * 16 + lanes
                pr = _shr_l(pv, 7)
                pc = pv & 127
                plsc.store_scatter(labo_v, [pr, pc], label)
                plsc.store_scatter(sco_v, [pr, pc], score)
                ob = (j * 16 + lanes) * 4
                obr = _shr_l(ob, 7)
                obc = ob & 127
                plsc.store_scatter(boxo_v, [obr, obc], (cx - 0.5 * w) * sx)
                plsc.store_scatter(boxo_v, [obr, obc + 1], (cy - 0.5 * h) * sy)
                plsc.store_scatter(boxo_v, [obr, obc + 2], (cx + 0.5 * w) * sx)
                plsc.store_scatter(boxo_v, [obr, obc + 3], (cy + 0.5 * h) * sy)
                return 0
            lax.fori_loop(0, KPAD // 16, out_body, 0)

            pltpu.sync_copy(labo_v, lab_hbm.at[r])
            pltpu.sync_copy(sco_v, sc_hbm.at[r])
            pltpu.sync_copy(boxo_v, box_hbm.at[r])
            return 0

        lax.fori_loop(0, rows_per_w, row_body, 0)

    return sc_call


_SC_CALL = None


def kernel(pred_logits, pred_boxes, orig_target_sizes):
    global _SC_CALL
    if _SC_CALL is None:
        _SC_CALL = _build_sc_call()
    logits3 = jnp.pad(
        pred_logits.reshape(B, N), ((0, 0), (0, NPAD - N)),
        constant_values=float("-inf")).reshape(B, MROW, 128)
    sizes_f = orig_target_sizes.astype(_f32)
    boxes_aug = jnp.concatenate(
        [pred_boxes.reshape(B, NQ * 4), sizes_f,
         jnp.zeros((B, BOXM * 128 - NQ * 4 - 2), _f32)],
        axis=1).reshape(B, BOXM, 128)
    lab_p, sc_p, box_p = _SC_CALL(logits3, boxes_aug)
    labels = lab_p.reshape(B, OUTM * 128)[:, :K]
    scores = sc_p.reshape(B, OUTM * 128)[:, :K]
    boxes = box_p.reshape(B, BOXOM * 128)[:, :4 * K].reshape(B, K, 4)
    return (labels, boxes, scores)


# speedup vs baseline: 10.7178x; 1.2741x over previous
import functools

import jax
import jax.numpy as jnp
from jax import lax
from jax.experimental import pallas as pl
from jax.experimental.pallas import tpu as pltpu
from jax.experimental.pallas import tpu_sc as plsc

NUM_CLASSES = 80
K = 300
B = 64
NQ = 900
N = NQ * NUM_CLASSES
MROW = 568
NPAD = MROW * 128
BOXM = 32
KPAD = 304
OUTM = 8
BOXOM = 16
CAP = 4096

_i32 = jnp.int32
_f32 = jnp.float32
_SIGN = -2147483648


def _monotonic_key(bits):
    m = lax.shift_right_arithmetic(bits, jnp.full(bits.shape, 31, _i32))
    return bits ^ (m | jnp.full(bits.shape, _SIGN, _i32))


def _shr_l(x, amt):
    return lax.shift_right_logical(x, jnp.full(x.shape, amt, _i32))


def _build_sc_call():
    nc, ns = 2, 16
    nw = nc * ns
    rows_per_w = B // nw
    mesh = plsc.VectorSubcoreMesh(core_axis_name="c", subcore_axis_name="s",
                                  num_cores=nc, num_subcores=ns)

    @functools.partial(
        pl.kernel,
        mesh=mesh,
        compiler_params=pltpu.CompilerParams(needs_layout_passes=False),
        out_type=[
            jax.ShapeDtypeStruct((B, OUTM, 128), _i32),
            jax.ShapeDtypeStruct((B, OUTM, 128), _f32),
            jax.ShapeDtypeStruct((B, BOXOM, 128), _f32),
        ],
        scratch_types=[
            pltpu.VMEM((MROW, 128), _f32),
            pltpu.VMEM((BOXM, 128), _f32),
            pltpu.VMEM((4 * 4096,), _i32),
            pltpu.VMEM((CAP,), _i32),
            pltpu.VMEM((CAP,), _i32),
            pltpu.VMEM((CAP,), _i32),
            pltpu.VMEM((CAP,), _i32),
            pltpu.VMEM((512,), _i32),
            pltpu.VMEM((OUTM, 128), _i32),
            pltpu.VMEM((OUTM, 128), _f32),
            pltpu.VMEM((BOXOM, 128), _f32),
        ],
    )
    def sc_call(logits_hbm, boxes_hbm, lab_hbm, sc_hbm, box_hbm,
                data_v, boxr_v, hist_v,
                skey_a, skey_b, sidx_a, sidx_b, cnt_v,
                labo_v, sco_v, boxo_v):
        wid = lax.axis_index("s") * nc + lax.axis_index("c")
        lanes = lax.iota(_i32, 16)
        ones = jnp.ones((16,), _i32)
        zeros16 = jnp.zeros((16,), _i32)

        def row_body(rr, _unused):
            r = wid * rows_per_w + rr
            pltpu.sync_copy(logits_hbm.at[r], data_v)
            pltpu.sync_copy(boxes_hbm.at[r], boxr_v)

            @plsc.parallel_loop(0, 128, 1, unroll=4)
            def _clr_hist(i):
                for k in range(8):
                    hist_v[pl.ds(i * 128 + k * 16, 16)] = zeros16

            @plsc.parallel_loop(0, MROW, 1, unroll=2)
            def _hist_body(i):
                row = data_v.at[i]
                for k in range(8):
                    x = row[pl.ds(k * 16, 16)]
                    key = _monotonic_key(lax.bitcast_convert_type(x, _i32))
                    bucket = _shr_l(key, 24)
                    hidx = bucket * 16 + lanes + (k % 4) * 4096
                    plsc.addupdate_scatter(hist_v, [hidx], ones)

            @plsc.parallel_loop(0, 64, 1, unroll=4)
            def _merge_hist(i):
                for k in range(4):
                    b = i * 4 + k
                    h = (hist_v[pl.ds(b * 16, 16)]
                         + hist_v[pl.ds(4096 + b * 16, 16)]
                         + hist_v[pl.ds(8192 + b * 16, 16)]
                         + hist_v[pl.ds(12288 + b * 16, 16)])
                    hist_v[pl.ds(b * 16, 16)] = h

            def scan_grp(i, carry):
                cum, gstar, gbase = carry
                g = 15 - i
                acc = hist_v[pl.ds(g * 256, 16)]
                for k in range(1, 16):
                    acc = acc + hist_v[pl.ds(g * 256 + k * 16, 16)]
                sg = jnp.sum(acc)
                newcum = cum + sg
                found = jnp.logical_and(cum < K, newcum >= K)
                gstar = jnp.where(found, g, gstar)
                gbase = jnp.where(found, cum, gbase)
                return (newcum, gstar, gbase)
            _, gstar, gbase = lax.fori_loop(
                0, 16, scan_grp, (_i32(0), _i32(0), _i32(0)))

            def scan_bck(i, carry):
                cum, bstar = carry
                b = gstar * 16 + 15 - i
                tot = jnp.sum(hist_v[pl.ds(b * 16, 16)])
                newcum = cum + tot
                found = jnp.logical_and(cum < K, newcum >= K)
                bstar = jnp.where(found, b, bstar)
                return (newcum, bstar)
            _, bstar = lax.fori_loop(0, 16, scan_bck, (gbase, _i32(0)))

            t = lax.shift_left(bstar, _i32(24))
            fbits = jnp.where(bstar >= 128, t & _i32(0x7FFFFFFF), ~t)
            thr = lax.bitcast_convert_type(
                lax.broadcast_in_dim(fbits, (16,), ()), _f32)
            capv = jnp.full((16,), CAP, _i32)

            @plsc.parallel_loop(0, MROW, 1, unroll=2,
                                carry=jnp.zeros((16,), _i32))
            def off_vec(i, off_vec):
                row = data_v.at[i]
                base = i * 128
                for k in range(8):
                    x = row[pl.ds(k * 16, 16)]
                    m = x >= thr
                    mi = jnp.where(m, ones, zeros16)
                    inc = plsc.cumsum(mi)
                    pos = off_vec + inc - mi
                    mm = jnp.logical_and(m, pos < capv)
                    plsc.store_scatter(sidx_a, [pos],
                                       base + k * 16 + lanes, mask=mm)
                    off_vec = off_vec + plsc.all_reduce_population_count(m)
                return off_vec
            n_d = jnp.minimum(jnp.max(off_vec), _i32(CAP))
            nvd = ((n_d + 63) // 64) * 4

            @plsc.parallel_loop(0, nvd // 4, 1, unroll=2)
            def _conv_body(jo):
                for kk in range(4):
                    j = jo * 4 + kk
                    iv = sidx_a[pl.ds(j * 16, 16)]
                    valid = (j * 16 + lanes) < n_d
                    ivs = jnp.where(valid, iv, 0)
                    xr = _shr_l(ivs, 7)
                    xc = ivs & 127
                    x = plsc.load_gather(data_v, [xr, xc])
                    ki = ~_monotonic_key(lax.bitcast_convert_type(x, _i32))
                    skey_a[pl.ds(j * 16, 16)] = jnp.where(valid, ki, _i32(-1))
                    sidx_a[pl.ds(j * 16, 16)] = ivs

            bufs = [(skey_a, sidx_a), (skey_b, sidx_b)]
            for p in range(7):
                src_k, src_i = bufs[p % 2]
                dst_k, dst_i = bufs[(p + 1) % 2]
                shift = 5 * p

                @plsc.parallel_loop(0, 8, 1, unroll=4)
                def _clr_cnt(i):
                    for k in range(4):
                        cnt_v[pl.ds(i * 64 + k * 16, 16)] = zeros16

                @plsc.parallel_loop(0, nvd // 4, 1, unroll=2)
                def _rs_hist(jo):
                    for kk in range(4):
                        j = jo * 4 + kk
                        g = lanes * nvd + j
                        kv = plsc.load_gather(src_k, [g])
                        d = _shr_l(kv, shift) & 31
                        plsc.addupdate_scatter(cnt_v, [d * 16 + lanes], ones)

                acc0 = jnp.zeros((16,), _i32)
                acc1 = jnp.zeros((16,), _i32)
                for k in range(16):
                    acc0 = acc0 + plsc.load_gather(cnt_v, [lanes * 16 + k])
                    acc1 = acc1 + plsc.load_gather(cnt_v,
                                                   [(lanes + 16) * 16 + k])
                c0 = plsc.cumsum(acc0)
                base0 = c0 - acc0
                tot0 = jnp.max(c0)
                c1 = plsc.cumsum(acc1)
                base1 = c1 - acc1 + tot0

                @plsc.parallel_loop(0, 8, 1, unroll=2)
                def _rs_lanepfx(do):
                    for kk in range(4):
                        d = do * 4 + kk
                        v = cnt_v[pl.ds(d * 16, 16)]
                        cnt_v[pl.ds(d * 16, 16)] = plsc.cumsum(v) - v
                for k in range(16):
                    plsc.addupdate_scatter(cnt_v, [lanes * 16 + k], base0)
                    plsc.addupdate_scatter(cnt_v, [(lanes + 16) * 16 + k],
                                           base1)

                def rs_perm(jo, _):
                    for kk in range(4):
                        j = jo * 4 + kk
                        g = lanes * nvd + j
                        kv = plsc.load_gather(src_k, [g])
                        iv = plsc.load_gather(src_i, [g])
                        d = _shr_l(kv, shift) & 31
                        ci = d * 16 + lanes
                        pos = plsc.load_gather(cnt_v, [ci])
                        plsc.store_scatter(cnt_v, [ci], pos + 1)
                        plsc.store_scatter(dst_k, [pos], kv)
                        plsc.store_scatter(dst_i, [pos], iv)
                    return 0
                lax.fori_loop(0, nvd // 4, rs_perm, 0)

            sc_row = boxr_v.at[28][pl.ds(16, 16)]
            sxs = jnp.sum(jnp.where(lanes == 0, sc_row, 0.0))
            sys_ = jnp.sum(jnp.where(lanes == 1, sc_row, 0.0))
            sx = lax.broadcast_in_dim(sxs, (16,), ())
            sy = lax.broadcast_in_dim(sys_, (16,), ())

            @plsc.parallel_loop(0, KPAD // 16, 1, unroll=2)
            def _out_body(j):
                ki = skey_b[pl.ds(j * 16, 16)]
                key = ~ki
                m = lax.shift_right_arithmetic(key, jnp.full((16,), 31, _i32))
                bits = key ^ (~m | jnp.full((16,), _SIGN, _i32))
                x = lax.bitcast_convert_type(bits, _f32)
                score = 1.0 / (1.0 + jnp.exp(-x))
                idxv = sidx_b[pl.ds(j * 16, 16)]
                q = idxv // NUM_CLASSES
                label = idxv - q * NUM_CLASSES
                q = jnp.minimum(jnp.maximum(q, 0), NQ - 1)
                qq = q * 4
                qr = _shr_l(qq, 7)
                qc = qq & 127
                cx = plsc.load_gather(boxr_v, [qr, qc])
                cy = plsc.load_gather(boxr_v, [qr, qc + 1])
                w = plsc.load_gather(boxr_v, [qr, qc + 2])
                h = plsc.load_gather(boxr_v, [qr, qc + 3])
                pv = j * 16 + lanes
                pr = _shr_l(pv, 7)
                pc = pv & 127
                plsc.store_scatter(labo_v, [pr, pc], label)
                plsc.store_scatter(sco_v, [pr, pc], score)
                ob = (j * 16 + lanes) * 4
                obr = _shr_l(ob, 7)
                obc = ob & 127
                plsc.store_scatter(boxo_v, [obr, obc], (cx - 0.5 * w) * sx)
                plsc.store_scatter(boxo_v, [obr, obc + 1], (cy - 0.5 * h) * sy)
                plsc.store_scatter(boxo_v, [obr, obc + 2], (cx + 0.5 * w) * sx)
                plsc.store_scatter(boxo_v, [obr, obc + 3], (cy + 0.5 * h) * sy)

            pltpu.sync_copy(labo_v, lab_hbm.at[r])
            pltpu.sync_copy(sco_v, sc_hbm.at[r])
            pltpu.sync_copy(boxo_v, box_hbm.at[r])
            return 0

        lax.fori_loop(0, rows_per_w, row_body, 0)

    return sc_call


_SC_CALL = None


def kernel(pred_logits, pred_boxes, orig_target_sizes):
    global _SC_CALL
    if _SC_CALL is None:
        _SC_CALL = _build_sc_call()
    logits3 = jnp.pad(
        pred_logits.reshape(B, N), ((0, 0), (0, NPAD - N)),
        constant_values=float("-inf")).reshape(B, MROW, 128)
    sizes_f = orig_target_sizes.astype(_f32)
    boxes_aug = jnp.concatenate(
        [pred_boxes.reshape(B, NQ * 4), sizes_f,
         jnp.zeros((B, BOXM * 128 - NQ * 4 - 2), _f32)],
        axis=1).reshape(B, BOXM, 128)
    lab_p, sc_p, box_p = _SC_CALL(logits3, boxes_aug)
    labels = lab_p.reshape(B, OUTM * 128)[:, :K]
    scores = sc_p.reshape(B, OUTM * 128)[:, :K]
    boxes = box_p.reshape(B, BOXOM * 128)[:, :4 * K].reshape(B, K, 4)
    return (labels, boxes, scores)
